# Initial kernel scaffold; baseline (speedup 1.0000x reference)
import jax
import jax.numpy as jnp
from jax.experimental import pallas as pl


def kernel(x, edge_index, edge_attr, batch, node_w, node_b, edge_w, edge_b, ln_g, ln_b, t, mlp_w1, mlp_b1, mlp_ln_g, mlp_ln_b, mlp_w2, mlp_b2, lin_w, lin_b):
    raise NotImplementedError("write your pallas kernel here")


# speedup vs baseline: 2.4993x; 2.4993x over previous
import functools

import jax
import jax.numpy as jnp
from jax import lax
from jax.experimental import pallas as pl
from jax.experimental.pallas import tpu as pltpu
from jax.experimental.pallas import tpu_sc as plsc

N = 10000
E = 320000
HID = 128
NG = 64
CHUNK = 128
NCHUNK = E // CHUNK
NSUB = 16
ROWS_PER_SUB = 624
NB = 400
EB = 1000


def _sc_edge_pass(r, e, src, dst, par, is_num):
    mesh = plsc.VectorSubcoreMesh(core_axis_name="c", subcore_axis_name="s",
                                  num_cores=1)

    @functools.partial(
        pl.kernel,
        out_type=jax.ShapeDtypeStruct((N, 128), jnp.float32),
        mesh=mesh,
        scratch_types=[
            pltpu.VMEM((CHUNK,), jnp.int32),
            pltpu.VMEM((1, CHUNK), jnp.int32),
            pltpu.VMEM((CHUNK, 128), jnp.float32),
            pltpu.VMEM((CHUNK, 128), jnp.float32),
            pltpu.VMEM((2, 16), jnp.float32),
            pltpu.VMEM((64, 128), jnp.float32),
            pltpu.VMEM_SHARED((N, 128), jnp.float32),
            pltpu.SemaphoreType.DMA,
        ],
    )
    def body(r_hbm, e_hbm, src_hbm, dst_hbm, par_hbm, out_hbm,
             src_v, dst_v, h_v, e_v, par_v, z_v, acc_sh, sem):
        s = lax.axis_index("s")

        pltpu.sync_copy(par_hbm, par_v)
        tvec = par_v[0, :]
        avec = par_v[1, :]

        zero = jnp.zeros((16,), jnp.float32)

        def zrow(i, carry):
            for q in range(8):
                z_v[i, pl.ds(q * 16, 16)] = zero
            return carry

        lax.fori_loop(0, 64, zrow, 0)
        base = s * ROWS_PER_SUB
        for j in range(9):
            pltpu.sync_copy(z_v, acc_sh.at[pl.ds(base + j * 64, 64)])
        pltpu.sync_copy(z_v.at[pl.ds(0, 48)], acc_sh.at[pl.ds(base + 576, 48)])

        @pl.when(s == NSUB - 1)
        def _():
            pltpu.sync_copy(z_v.at[pl.ds(0, 16)],
                            acc_sh.at[pl.ds(NSUB * ROWS_PER_SUB, 16)])

        plsc.subcore_barrier()

        nloop = (NCHUNK + NSUB - 1) // NSUB

        def chunk(j, carry):
            k = j * NSUB + s

            @pl.when(k < NCHUNK)
            def _():
                pltpu.sync_copy(src_hbm.at[k], src_v)
                pltpu.sync_copy(dst_hbm.at[k], dst_v.at[0])
                cp = pltpu.async_copy(r_hbm.at[src_v], h_v, sem)
                pltpu.sync_copy(e_hbm.at[pl.ds(k * CHUNK, CHUNK)], e_v)
                cp.wait()

                def row(i, rc):
                    for q in range(8):
                        hh = h_v[i, pl.ds(q * 16, 16)]
                        ee = e_v[i, pl.ds(q * 16, 16)]
                        msg = jnp.maximum(hh + ee, 0.0) + 1e-7
                        ex = jnp.exp(msg * tvec - avec)
                        e_v[i, pl.ds(q * 16, 16)] = msg * ex if is_num else ex
                    return rc

                lax.fori_loop(0, CHUNK, row, 0)
                pltpu.sync_copy(e_v, acc_sh.at[dst_v.at[0]], add=True)

            return carry

        lax.fori_loop(0, nloop, chunk, 0)
        plsc.subcore_barrier()

        for j in range(4):
            pltpu.sync_copy(acc_sh.at[pl.ds(base + j * 128, 128)],
                            out_hbm.at[pl.ds(base + j * 128, 128)])
        pltpu.sync_copy(acc_sh.at[pl.ds(base + 512, 112)],
                        out_hbm.at[pl.ds(base + 512, 112)])

        @pl.when(s == NSUB - 1)
        def _():
            pltpu.sync_copy(acc_sh.at[pl.ds(NSUB * ROWS_PER_SUB, 16)],
                            out_hbm.at[pl.ds(NSUB * ROWS_PER_SUB, 16)])

    return body(r, e, src, dst, par)


def _edge_linear_body(ea_ref, w_ref, b_ref, e_ref, mx_ref):
    e = jnp.dot(ea_ref[...], w_ref[...], preferred_element_type=jnp.float32)
    e = e + b_ref[...]
    e_ref[...] = e

    @pl.when(pl.program_id(0) == 0)
    def _():
        mx_ref[0, 0] = -1e30

    mx_ref[0, 0] = jnp.maximum(mx_ref[0, 0], jnp.max(e))


def _edge_linear(edge_attr, edge_w, edge_b):
    grid = E // EB
    return pl.pallas_call(
        _edge_linear_body,
        grid=(grid,),
        in_specs=[
            pl.BlockSpec((EB, 16), lambda i: (i, 0)),
            pl.BlockSpec((16, 128), lambda i: (0, 0)),
            pl.BlockSpec((1, 128), lambda i: (0, 0)),
        ],
        out_specs=[
            pl.BlockSpec((EB, 128), lambda i: (i, 0)),
            pl.BlockSpec(memory_space=pltpu.SMEM, block_shape=(1, 1), index_map=lambda i: (0, 0)),
        ],
        out_shape=[
            jax.ShapeDtypeStruct((E, 128), jnp.float32),
            jax.ShapeDtypeStruct((1, 1), jnp.float32),
        ],
    )(edge_attr, edge_w, edge_b.reshape(1, 128))


def _node_linear_body(x_ref, w_ref, b_ref, h_ref, mx_ref):
    h = jnp.dot(x_ref[...], w_ref[...], preferred_element_type=jnp.float32)
    h = h + b_ref[...]
    h_ref[...] = h

    @pl.when(pl.program_id(0) == 0)
    def _():
        mx_ref[0, 0] = -1e30

    mx_ref[0, 0] = jnp.maximum(mx_ref[0, 0], jnp.max(h))


def _node_linear(x, node_w, node_b):
    grid = N // NB
    return pl.pallas_call(
        _node_linear_body,
        grid=(grid,),
        in_specs=[
            pl.BlockSpec((NB, 128), lambda i: (i, 0)),
            pl.BlockSpec((128, 128), lambda i: (0, 0)),
            pl.BlockSpec((1, 128), lambda i: (0, 0)),
        ],
        out_specs=[
            pl.BlockSpec((NB, 128), lambda i: (i, 0)),
            pl.BlockSpec(memory_space=pltpu.SMEM, block_shape=(1, 1), index_map=lambda i: (0, 0)),
        ],
        out_shape=[
            jax.ShapeDtypeStruct((N, 128), jnp.float32),
            jax.ShapeDtypeStruct((1, 1), jnp.float32),
        ],
    )(x, node_w, node_b.reshape(1, 128))


def _ln(h, g, b):
    mu = jnp.mean(h, axis=-1, keepdims=True)
    var = jnp.mean((h - mu) ** 2, axis=-1, keepdims=True)
    return (h - mu) * lax.rsqrt(var + 1e-5) * g + b


def _layer_body(first, num_ref, den_ref, r_ref, hprev_ref, w1_ref, b1_ref,
                g1_ref, c1_ref, w2_ref, b2_ref, gn_ref, bn_ref, hn_ref,
                rf_ref, mx_ref):
    out = num_ref[...] / (den_ref[...] + 1e-16) + r_ref[...]
    z = jnp.dot(out, w1_ref[...], preferred_element_type=jnp.float32) + b1_ref[...]
    z = _ln(z, g1_ref[...], c1_ref[...])
    z = jnp.maximum(z, 0.0)
    z = jnp.dot(z, w2_ref[...], preferred_element_type=jnp.float32) + b2_ref[...]
    if first:
        hn = z
    else:
        hn = hprev_ref[...] + z
    rn = jnp.maximum(_ln(hn, gn_ref[...], bn_ref[...]), 0.0)
    hn_ref[...] = hn
    rf_ref[...] = rn

    @pl.when(pl.program_id(0) == 0)
    def _():
        mx_ref[0, 0] = -1e30

    mx_ref[0, 0] = jnp.maximum(mx_ref[0, 0], jnp.max(rn))


def _layer_tc(num, den, r, hprev, w1, b1, g1, c1, w2, b2, gn, bn, first):
    grid = N // NB
    return pl.pallas_call(
        functools.partial(_layer_body, first),
        grid=(grid,),
        in_specs=[
            pl.BlockSpec((NB, 128), lambda i: (i, 0)),
            pl.BlockSpec((NB, 128), lambda i: (i, 0)),
            pl.BlockSpec((NB, 128), lambda i: (i, 0)),
            pl.BlockSpec((NB, 128), lambda i: (i, 0)),
            pl.BlockSpec((128, 256), lambda i: (0, 0)),
            pl.BlockSpec((1, 256), lambda i: (0, 0)),
            pl.BlockSpec((1, 256), lambda i: (0, 0)),
            pl.BlockSpec((1, 256), lambda i: (0, 0)),
            pl.BlockSpec((256, 128), lambda i: (0, 0)),
            pl.BlockSpec((1, 128), lambda i: (0, 0)),
            pl.BlockSpec((1, 128), lambda i: (0, 0)),
            pl.BlockSpec((1, 128), lambda i: (0, 0)),
        ],
        out_specs=[
            pl.BlockSpec((NB, 128), lambda i: (i, 0)),
            pl.BlockSpec((NB, 128), lambda i: (i, 0)),
            pl.BlockSpec(memory_space=pltpu.SMEM, block_shape=(1, 1), index_map=lambda i: (0, 0)),
        ],
        out_shape=[
            jax.ShapeDtypeStruct((N, 128), jnp.float32),
            jax.ShapeDtypeStruct((N, 128), jnp.float32),
            jax.ShapeDtypeStruct((1, 1), jnp.float32),
        ],
    )(num, den, r, hprev, w1, b1.reshape(1, 256), g1.reshape(1, 256),
      c1.reshape(1, 256), w2, b2.reshape(1, 128), gn.reshape(1, 128),
      bn.reshape(1, 128))


def _pool_body(r_ref, lw_ref, lb_ref, batch_ref, out_ref, acc_ref):
    i = pl.program_id(0)

    @pl.when(i == 0)
    def _():
        acc_ref[...] = jnp.zeros_like(acc_ref)

    o = jnp.dot(r_ref[...], lw_ref[...], preferred_element_type=jnp.float32)
    o = o + lb_ref[0, 0]
    bb = batch_ref[0, 0, :]
    gid = lax.broadcasted_iota(jnp.int32, (NG, NB), 0)
    onehot = (gid == bb[None, :]).astype(jnp.float32)
    acc_ref[:, 0:1] = acc_ref[:, 0:1] + jnp.dot(
        onehot, o, preferred_element_type=jnp.float32)
    acc_ref[:, 1:2] = acc_ref[:, 1:2] + jnp.sum(onehot, axis=1, keepdims=True)

    @pl.when(i == pl.num_programs(0) - 1)
    def _():
        out_ref[...] = acc_ref[:, 0:1] / jnp.maximum(acc_ref[:, 1:2], 1.0)


def _pool_tc(r4, lin_w, lin_b, batch):
    grid = N // NB
    return pl.pallas_call(
        _pool_body,
        grid=(grid,),
        in_specs=[
            pl.BlockSpec((NB, 128), lambda i: (i, 0)),
            pl.BlockSpec((128, 1), lambda i: (0, 0)),
            pl.BlockSpec(memory_space=pltpu.SMEM, block_shape=(1, 1), index_map=lambda i: (0, 0)),
            pl.BlockSpec((1, 1, NB), lambda i: (i, 0, 0)),
        ],
        out_specs=pl.BlockSpec((NG, 1), lambda i: (0, 0)),
        out_shape=jax.ShapeDtypeStruct((NG, 1), jnp.float32),
        scratch_shapes=[pltpu.VMEM((NG, 2), jnp.float32)],
    )(r4, lin_w, lin_b.reshape(1, 1), batch.reshape(N // NB, 1, NB))


def _params_vec(t_i, maxr, maxe):
    mmsg = jnp.maximum(maxr + maxe, 0.0) + 1e-7
    a = jnp.maximum(t_i * mmsg, t_i * 1e-7)
    p = jnp.stack([jnp.full((16,), t_i, jnp.float32),
                   jnp.full((16,), a, jnp.float32)])
    return p


def kernel(x, edge_index, edge_attr, batch, node_w, node_b, edge_w, edge_b,
           ln_g, ln_b, t, mlp_w1, mlp_b1, mlp_ln_g, mlp_ln_b, mlp_w2, mlp_b2,
           lin_w, lin_b):
    src = edge_index[0].reshape(NCHUNK, CHUNK)
    dst = edge_index[1].reshape(NCHUNK, CHUNK)

    e, maxe = _edge_linear(edge_attr, edge_w, edge_b)
    maxe = maxe[0, 0]

    h0, maxr = _node_linear(x, node_w, node_b)
    maxr = maxr[0, 0]
    r_full = h0
    h_state = h0

    L = t.shape[0]
    for i in range(L):
        par = _params_vec(t[i], maxr, maxe)
        num = _sc_edge_pass(r_full, e, src, dst, par, True)
        den = _sc_edge_pass(r_full, e, src, dst, par, False)
        nxt = (i + 1) % L
        h_state, r_full, maxr = _layer_tc(
            num, den, r_full, h_state, mlp_w1[i], mlp_b1[i], mlp_ln_g[i],
            mlp_ln_b[i], mlp_w2[i], mlp_b2[i], ln_g[nxt], ln_b[nxt],
            first=(i == 0))
        maxr = maxr[0, 0]

    return _pool_tc(r_full, lin_w, lin_b, batch)
